# baseline (speedup 1.0000x reference)
import functools

import jax
import jax.numpy as jnp
from jax import lax
from jax.experimental import pallas as pl
from jax.experimental.pallas import tpu as pltpu
from jax.experimental.pallas import tpu_sc as plsc

_EPS = 1e-5
_NC = 2
_NS = 16
_LANES = 16


def _xw_body(nrels, x_ref, w_ref, bres_ref, out_ref):
    r = pl.program_id(0)
    t = jnp.dot(x_ref[...], w_ref[0], preferred_element_type=jnp.float32)
    t = jnp.where(r == nrels, jnp.maximum(t + bres_ref[...], 0.0), t)
    out_ref[0] = t


def _compute_xw(x, w_all, bres, nrels, block_n):
    n, d_in = x.shape
    rp1, _, d_out = w_all.shape
    grid = (rp1, n // block_n)
    return pl.pallas_call(
        functools.partial(_xw_body, nrels),
        grid=grid,
        in_specs=[
            pl.BlockSpec((block_n, d_in), lambda r, b: (b, 0)),
            pl.BlockSpec((1, d_in, d_out), lambda r, b: (r, 0, 0)),
            pl.BlockSpec((1, d_out), lambda r, b: (0, 0)),
        ],
        out_specs=pl.BlockSpec((1, block_n, d_out), lambda r, b: (r, b, 0)),
        out_shape=jax.ShapeDtypeStruct((rp1, n, d_out), jnp.float32),
    )(x, w_all, bres.reshape(1, d_out))


def _gidx_body(n, et_ref, src_ref, out_ref):
    out_ref[...] = et_ref[...] * n + src_ref[...]


def _compute_gidx(etype, src, n):
    e = etype.shape[0]
    cols = 128
    rows = e // cols
    out = pl.pallas_call(
        functools.partial(_gidx_body, n),
        out_shape=jax.ShapeDtypeStruct((rows, cols), jnp.int32),
    )(etype.reshape(rows, cols), src.reshape(rows, cols))
    return out.reshape(e)


def _make_sc_edge_kernel(n, d, e_pad, chunk):
    nw = _NC * _NS
    epw = e_pad // nw
    steps = epw // chunk
    assert steps % 2 == 0
    unit = 40
    units = n // unit
    rounds = (units + _NS - 1) // _NS

    def body(gidx_hbm, dst_hbm, norm_hbm, xw_hbm, out_hbm,
             iall_v, rows0_v, rows1_v, dbuf0_v, dbuf1_v, nbuf0_v, nbuf1_v,
             zbuf_v, agg_sh, semg0, semg1, semd0, semd1, sems0, sems1,
             semn0, semn1):
        c = lax.axis_index("c")
        s = lax.axis_index("s")

        zv = jnp.zeros((_LANES,), jnp.float32)

        def zrow(i, _):
            for j in range(d // _LANES):
                zbuf_v[i, pl.ds(j * _LANES, _LANES)] = zv
            return 0

        lax.fori_loop(0, unit, zrow, 0)
        for k in range(rounds):
            u = s + k * _NS

            @pl.when(u < units)
            def _():
                pltpu.sync_copy(zbuf_v, agg_sh.at[pl.ds(u * unit, unit)])
        plsc.subcore_barrier()

        wid = s * _NC + c
        off0 = wid * epw
        pltpu.sync_copy(gidx_hbm.at[pl.ds(off0, epw)], iall_v)

        def start_gather(t, buf, sem):
            pltpu.async_copy(
                xw_hbm.at[iall_v.at[pl.ds(t * chunk, chunk)]], buf, sem)

        def wait_gather(buf, sem):
            pltpu.make_async_copy(xw_hbm.at[pl.ds(0, chunk)], buf, sem).wait()

        def start_dst(t, dbuf, sem):
            pltpu.async_copy(dst_hbm.at[pl.ds(off0 + t * chunk, chunk)],
                             dbuf, sem)

        def start_norm(t, nbuf, sem):
            pltpu.async_copy(norm_hbm.at[pl.ds(off0 + t * chunk, chunk)],
                             nbuf, sem)

        def wait_norm(nbuf, sem):
            pltpu.make_async_copy(norm_hbm.at[pl.ds(0, chunk)], nbuf,
                                  sem).wait()

        def wait_dst(dbuf, sem):
            pltpu.make_async_copy(dst_hbm.at[pl.ds(0, chunk)], dbuf,
                                  sem).wait()

        def scale(nbuf, buf):
            for g in range(chunk // _LANES):
                nv = nbuf[pl.ds(g * _LANES, _LANES)]
                for i in range(_LANES):
                    ee = g * _LANES + i
                    nb = nv[i]
                    for j in range(d // _LANES):
                        sl = pl.ds(j * _LANES, _LANES)
                        buf[ee, sl] = buf[ee, sl] * nb

        def start_scatter(buf, dbuf, sem):
            pltpu.async_copy(buf, agg_sh.at[dbuf], sem, add=True)

        def wait_scatter(buf, dbuf, sem):
            pltpu.make_async_copy(buf, agg_sh.at[dbuf], sem).wait()

        start_dst(0, dbuf0_v, semd0)
        start_dst(1, dbuf1_v, semd1)
        start_norm(0, nbuf0_v, semn0)
        start_norm(1, nbuf1_v, semn1)
        start_gather(0, rows0_v, semg0)
        start_gather(1, rows1_v, semg1)

        def step2(p, _):
            t0 = 2 * p
            wait_gather(rows0_v, semg0)
            wait_norm(nbuf0_v, semn0)
            scale(nbuf0_v, rows0_v)
            wait_dst(dbuf0_v, semd0)
            start_scatter(rows0_v, dbuf0_v, sems0)

            wait_gather(rows1_v, semg1)
            wait_norm(nbuf1_v, semn1)
            scale(nbuf1_v, rows1_v)
            wait_dst(dbuf1_v, semd1)
            start_scatter(rows1_v, dbuf1_v, sems1)

            wait_scatter(rows0_v, dbuf0_v, sems0)

            @pl.when(t0 + 2 < steps)
            def _():
                start_dst(t0 + 2, dbuf0_v, semd0)
                start_norm(t0 + 2, nbuf0_v, semn0)
                start_gather(t0 + 2, rows0_v, semg0)

            wait_scatter(rows1_v, dbuf1_v, sems1)

            @pl.when(t0 + 3 < steps)
            def _():
                start_dst(t0 + 3, dbuf1_v, semd1)
                start_norm(t0 + 3, nbuf1_v, semn1)
                start_gather(t0 + 3, rows1_v, semg1)

            return 0

        lax.fori_loop(0, steps // 2, step2, 0)
        plsc.subcore_barrier()

        for k in range(rounds):
            u = s + k * _NS

            @pl.when(u < units)
            def _():
                pltpu.sync_copy(agg_sh.at[pl.ds(u * unit, unit)], zbuf_v)
                pltpu.sync_copy(zbuf_v, out_hbm.at[pl.ds(c * n + u * unit,
                                                         unit)])

    mesh = plsc.VectorSubcoreMesh(core_axis_name="c", subcore_axis_name="s")
    return pl.kernel(
        body,
        out_type=jax.ShapeDtypeStruct((_NC * n, d), jnp.float32),
        mesh=mesh,
        scratch_types=[
            pltpu.VMEM((epw,), jnp.int32),
            pltpu.VMEM((chunk, d), jnp.float32),
            pltpu.VMEM((chunk, d), jnp.float32),
            pltpu.VMEM((chunk,), jnp.int32),
            pltpu.VMEM((chunk,), jnp.int32),
            pltpu.VMEM((chunk,), jnp.float32),
            pltpu.VMEM((chunk,), jnp.float32),
            pltpu.VMEM((unit, d), jnp.float32),
            pltpu.VMEM_SHARED((n, d), jnp.float32),
            pltpu.SemaphoreType.DMA,
            pltpu.SemaphoreType.DMA,
            pltpu.SemaphoreType.DMA,
            pltpu.SemaphoreType.DMA,
            pltpu.SemaphoreType.DMA,
            pltpu.SemaphoreType.DMA,
            pltpu.SemaphoreType.DMA,
            pltpu.SemaphoreType.DMA,
        ],
    )


def _bn_body(n, eps, part_ref, res_ref, bias_ref, gamma_ref, beta_ref,
             out_ref):
    agg = part_ref[:n] + part_ref[n:]
    h = jnp.maximum(agg + bias_ref[...], 0.0)
    new = h + res_ref[...]
    mean = jnp.mean(new, axis=0, keepdims=True)
    var = jnp.mean((new - mean) * (new - mean), axis=0, keepdims=True)
    inv = lax.rsqrt(var + eps)
    out_ref[...] = (new - mean) * (inv * gamma_ref[...]) + beta_ref[...]


def _combine_bn(part, res, bias, gamma, beta, n, d):
    return pl.pallas_call(
        functools.partial(_bn_body, n, _EPS),
        out_shape=jax.ShapeDtypeStruct((n, d), jnp.float32),
    )(part, res, bias.reshape(1, d), gamma.reshape(1, d), beta.reshape(1, d))


def kernel(node_feats, edge_index, etype, norm, W, bias, Wres, bres, gamma,
           beta):
    n, d_in = node_feats.shape
    nrels, _, d_out = W.shape
    e = etype.shape[0]

    src = edge_index[0]
    dst = edge_index[1]

    w_all = jnp.concatenate([W, Wres[None]], axis=0)
    xw = _compute_xw(node_feats, w_all, bres, nrels, 2000)
    xw_flat = xw.reshape((nrels + 1) * n, d_out)
    res = xw_flat[nrels * n:]

    gidx = _compute_gidx(etype, src, n)

    chunk = 128
    nw = _NC * _NS
    quantum = nw * chunk * 2
    e_pad = ((e + quantum - 1) // quantum) * quantum
    pad = e_pad - e
    gidx_p = jnp.pad(gidx, (0, pad))
    dst_p = jnp.pad(dst, (0, pad))
    norm_p = jnp.pad(norm.reshape(e), (0, pad))

    sc = _make_sc_edge_kernel(n, d_out, e_pad, chunk)
    part = sc(gidx_p, dst_p, norm_p, xw_flat)

    return _combine_bn(part, res, bias, gamma, beta, n, d_out)


# speedup vs baseline: 1.0099x; 1.0099x over previous
import functools

import jax
import jax.numpy as jnp
from jax import lax
from jax.experimental import pallas as pl
from jax.experimental.pallas import tpu as pltpu
from jax.experimental.pallas import tpu_sc as plsc

_EPS = 1e-5
_NC = 2
_NS = 16
_LANES = 16


def _xw_body(nrels, x_ref, w_ref, bres_ref, out_ref):
    r = pl.program_id(0)
    t = jnp.dot(x_ref[...], w_ref[0], preferred_element_type=jnp.float32)
    t = jnp.where(r == nrels, jnp.maximum(t + bres_ref[...], 0.0), t)
    out_ref[0] = t


def _compute_xw(x, w_all, bres, nrels, block_n):
    n, d_in = x.shape
    rp1, _, d_out = w_all.shape
    grid = (rp1, n // block_n)
    return pl.pallas_call(
        functools.partial(_xw_body, nrels),
        grid=grid,
        in_specs=[
            pl.BlockSpec((block_n, d_in), lambda r, b: (b, 0)),
            pl.BlockSpec((1, d_in, d_out), lambda r, b: (r, 0, 0)),
            pl.BlockSpec((1, d_out), lambda r, b: (0, 0)),
        ],
        out_specs=pl.BlockSpec((1, block_n, d_out), lambda r, b: (r, b, 0)),
        out_shape=jax.ShapeDtypeStruct((rp1, n, d_out), jnp.float32),
    )(x, w_all, bres.reshape(1, d_out))


def _gidx_body(n, et_ref, src_ref, out_ref):
    out_ref[...] = et_ref[...] * n + src_ref[...]


def _compute_gidx(etype, src, n):
    e = etype.shape[0]
    cols = 128
    rows = e // cols
    out = pl.pallas_call(
        functools.partial(_gidx_body, n),
        out_shape=jax.ShapeDtypeStruct((rows, cols), jnp.int32),
    )(etype.reshape(rows, cols), src.reshape(rows, cols))
    return out.reshape(e)


def _make_sc_edge_kernel(n, d, e_pad, chunk):
    nw = _NC * _NS
    epw = e_pad // nw
    steps = epw // chunk
    assert steps % 4 == 0
    unit = 40
    units = n // unit
    rounds = (units + _NS - 1) // _NS

    depth = 4
    assert steps % depth == 0

    def body(gidx_hbm, dst_hbm, norm_hbm, xw_hbm, out_hbm,
             iall_v, r0, r1, r2, r3, d0, d1, d2, d3, n0, n1, n2, n3,
             zbuf_v, agg_sh, *sems):
        rows = [r0, r1, r2, r3]
        dbuf = [d0, d1, d2, d3]
        nbuf = [n0, n1, n2, n3]
        semg = sems[0:4]
        semd = sems[4:8]
        semn = sems[8:12]
        sems = sems[12:16]
        c = lax.axis_index("c")
        s = lax.axis_index("s")

        zv = jnp.zeros((_LANES,), jnp.float32)

        def zrow(i, _):
            for j in range(d // _LANES):
                zbuf_v[i, pl.ds(j * _LANES, _LANES)] = zv
            return 0

        lax.fori_loop(0, unit, zrow, 0)
        for k in range(rounds):
            u = s + k * _NS

            @pl.when(u < units)
            def _():
                pltpu.sync_copy(zbuf_v, agg_sh.at[pl.ds(u * unit, unit)])
        plsc.subcore_barrier()

        wid = s * _NC + c
        off0 = wid * epw
        pltpu.sync_copy(gidx_hbm.at[pl.ds(off0, epw)], iall_v)

        def start_gather(t, buf, sem):
            pltpu.async_copy(
                xw_hbm.at[iall_v.at[pl.ds(t * chunk, chunk)]], buf, sem)

        def wait_gather(buf, sem):
            pltpu.make_async_copy(xw_hbm.at[pl.ds(0, chunk)], buf, sem).wait()

        def start_dst(t, dbuf_, sem):
            pltpu.async_copy(dst_hbm.at[pl.ds(off0 + t * chunk, chunk)],
                             dbuf_, sem)

        def wait_dst(dbuf_, sem):
            pltpu.make_async_copy(dst_hbm.at[pl.ds(0, chunk)], dbuf_,
                                  sem).wait()

        def start_norm(t, nbuf_, sem):
            pltpu.async_copy(norm_hbm.at[pl.ds(off0 + t * chunk, chunk)],
                             nbuf_, sem)

        def wait_norm(nbuf_, sem):
            pltpu.make_async_copy(norm_hbm.at[pl.ds(0, chunk)], nbuf_,
                                  sem).wait()

        def scale(nbuf_, buf):
            for g in range(chunk // _LANES):
                nv = nbuf_[pl.ds(g * _LANES, _LANES)]
                for i in range(_LANES):
                    ee = g * _LANES + i
                    nb = nv[i]
                    for j in range(d // _LANES):
                        sl = pl.ds(j * _LANES, _LANES)
                        buf[ee, sl] = buf[ee, sl] * nb

        def start_scatter(buf, dbuf_, sem):
            pltpu.async_copy(buf, agg_sh.at[dbuf_], sem, add=True)

        def wait_scatter(buf, dbuf_, sem):
            pltpu.make_async_copy(buf, agg_sh.at[dbuf_], sem).wait()

        for q in range(depth):
            start_dst(q, dbuf[q], semd[q])
            start_norm(q, nbuf[q], semn[q])
            start_gather(q, rows[q], semg[q])

        def stepk(p, _):
            t0 = depth * p
            for q in range(depth):
                wait_gather(rows[q], semg[q])
                wait_norm(nbuf[q], semn[q])
                scale(nbuf[q], rows[q])
                wait_dst(dbuf[q], semd[q])
                start_scatter(rows[q], dbuf[q], sems[q])
            for q in range(depth):
                wait_scatter(rows[q], dbuf[q], sems[q])

                @pl.when(t0 + q + depth < steps)
                def _():
                    start_dst(t0 + q + depth, dbuf[q], semd[q])
                    start_norm(t0 + q + depth, nbuf[q], semn[q])
                    start_gather(t0 + q + depth, rows[q], semg[q])
            return 0

        lax.fori_loop(0, steps // depth, stepk, 0)
        plsc.subcore_barrier()

        for k in range(rounds):
            u = s + k * _NS

            @pl.when(u < units)
            def _():
                pltpu.sync_copy(agg_sh.at[pl.ds(u * unit, unit)], zbuf_v)
                pltpu.sync_copy(zbuf_v, out_hbm.at[pl.ds(c * n + u * unit,
                                                         unit)])

    mesh = plsc.VectorSubcoreMesh(core_axis_name="c", subcore_axis_name="s")
    return pl.kernel(
        body,
        out_type=jax.ShapeDtypeStruct((_NC * n, d), jnp.float32),
        mesh=mesh,
        scratch_types=(
            [pltpu.VMEM((epw,), jnp.int32)]
            + [pltpu.VMEM((chunk, d), jnp.float32)] * 4
            + [pltpu.VMEM((chunk,), jnp.int32)] * 4
            + [pltpu.VMEM((chunk,), jnp.float32)] * 4
            + [pltpu.VMEM((unit, d), jnp.float32),
               pltpu.VMEM_SHARED((n, d), jnp.float32)]
            + [pltpu.SemaphoreType.DMA] * 16
        ),
    )


def _bn_body(n, eps, part_ref, res_ref, bias_ref, gamma_ref, beta_ref,
             out_ref):
    agg = part_ref[:n] + part_ref[n:]
    h = jnp.maximum(agg + bias_ref[...], 0.0)
    new = h + res_ref[...]
    mean = jnp.mean(new, axis=0, keepdims=True)
    var = jnp.mean((new - mean) * (new - mean), axis=0, keepdims=True)
    inv = lax.rsqrt(var + eps)
    out_ref[...] = (new - mean) * (inv * gamma_ref[...]) + beta_ref[...]


def _combine_bn(part, res, bias, gamma, beta, n, d):
    return pl.pallas_call(
        functools.partial(_bn_body, n, _EPS),
        out_shape=jax.ShapeDtypeStruct((n, d), jnp.float32),
    )(part, res, bias.reshape(1, d), gamma.reshape(1, d), beta.reshape(1, d))


def kernel(node_feats, edge_index, etype, norm, W, bias, Wres, bres, gamma,
           beta):
    n, d_in = node_feats.shape
    nrels, _, d_out = W.shape
    e = etype.shape[0]

    src = edge_index[0]
    dst = edge_index[1]

    w_all = jnp.concatenate([W, Wres[None]], axis=0)
    xw = _compute_xw(node_feats, w_all, bres, nrels, 2000)
    xw_flat = xw.reshape((nrels + 1) * n, d_out)
    res = xw_flat[nrels * n:]

    gidx = _compute_gidx(etype, src, n)

    chunk = 64
    nw = _NC * _NS
    quantum = nw * chunk * 4
    e_pad = ((e + quantum - 1) // quantum) * quantum
    pad = e_pad - e
    gidx_p = jnp.pad(gidx, (0, pad))
    dst_p = jnp.pad(dst, (0, pad))
    norm_p = jnp.pad(norm.reshape(e), (0, pad))

    sc = _make_sc_edge_kernel(n, d_out, e_pad, chunk)
    part = sc(gidx_p, dst_p, norm_p, xw_flat)

    return _combine_bn(part, res, bias, gamma, beta, n, d_out)


# speedup vs baseline: 1.8144x; 1.7966x over previous
import functools

import jax
import jax.numpy as jnp
from jax import lax
from jax.experimental import pallas as pl
from jax.experimental.pallas import tpu as pltpu
from jax.experimental.pallas import tpu_sc as plsc

_EPS = 1e-5
_NC = 2
_NS = 16
_LANES = 16


def _xw_body(nrels, x_ref, w_ref, bres_ref, out_ref):
    r = pl.program_id(1)
    t = jnp.dot(x_ref[...], w_ref[0], preferred_element_type=jnp.float32)
    t = jnp.where(r == nrels, jnp.maximum(t + bres_ref[...], 0.0), t)
    out_ref[0] = t


def _compute_xw(x, w_all, bres, nrels, block_n):
    n, d_in = x.shape
    rp1, _, d_out = w_all.shape
    grid = (n // block_n, rp1)
    return pl.pallas_call(
        functools.partial(_xw_body, nrels),
        grid=grid,
        in_specs=[
            pl.BlockSpec((block_n, d_in), lambda b, r: (b, 0)),
            pl.BlockSpec((1, d_in, d_out), lambda b, r: (r, 0, 0)),
            pl.BlockSpec((1, d_out), lambda b, r: (0, 0)),
        ],
        out_specs=pl.BlockSpec((1, block_n, d_out), lambda b, r: (r, b, 0)),
        out_shape=jax.ShapeDtypeStruct((rp1, n, d_out), jnp.float32),
    )(x, w_all, bres.reshape(1, d_out))


def _gidx_body(n, et_ref, src_ref, out_ref):
    out_ref[...] = et_ref[...] * n + src_ref[...]


def _compute_gidx(etype, src, n):
    e = etype.shape[0]
    cols = 128
    rows = e // cols
    out = pl.pallas_call(
        functools.partial(_gidx_body, n),
        out_shape=jax.ShapeDtypeStruct((rows, cols), jnp.int32),
    )(etype.reshape(rows, cols), src.reshape(rows, cols))
    return out.reshape(e)


def _make_sc_edge_kernel(n, d, e_pad, chunk, frac0):
    s_total = e_pad // (_NS * chunk)
    steps0 = 2 * int(round(s_total * frac0 / 2.0))
    steps1 = s_total - steps0
    assert steps0 % 2 == 0 and steps1 % 2 == 0 and steps1 >= 2
    epw0 = steps0 * chunk
    epw1 = steps1 * chunk
    unit = 40
    units = n // unit
    rounds = (units + _NS - 1) // _NS

    def body(gidx_hbm, dst_hbm, norm_hbm, xw_hbm, out_hbm,
             iall_v, nall_v, rows0_v, rows1_v, dbuf0_v, dbuf1_v,
             agg_sh, semg0, semg1, semd0, semd1, sems0, sems1, semz):
        c = lax.axis_index("c")
        s = lax.axis_index("s")

        zv = jnp.zeros((_LANES,), jnp.float32)

        def zrow(i, _):
            for j in range(d // _LANES):
                rows0_v[i, pl.ds(j * _LANES, _LANES)] = zv
            return 0

        lax.fori_loop(0, unit, zrow, 0)
        for k in range(rounds):
            u = s + k * _NS

            @pl.when(u < units)
            def _():
                pltpu.async_copy(rows0_v.at[pl.ds(0, unit)],
                                 agg_sh.at[pl.ds(u * unit, unit)], semz)
        for k in range(rounds):
            u = s + k * _NS

            @pl.when(u < units)
            def _():
                pltpu.make_async_copy(rows0_v.at[pl.ds(0, unit)],
                                      agg_sh.at[pl.ds(0, unit)],
                                      semz).wait()
        plsc.subcore_barrier()

        off0 = jnp.where(c == 0, s * epw0, _NS * epw0 + s * epw1)
        steps = jnp.where(c == 0, steps0, steps1)

        @pl.when(c == 0)
        def _():
            pltpu.async_copy(gidx_hbm.at[pl.ds(off0, epw0)], iall_v, semz)
            pltpu.async_copy(norm_hbm.at[pl.ds(off0, epw0)], nall_v, semz)
            pltpu.make_async_copy(gidx_hbm.at[pl.ds(off0, epw0)], iall_v,
                                  semz).wait()
            pltpu.make_async_copy(norm_hbm.at[pl.ds(off0, epw0)], nall_v,
                                  semz).wait()

        @pl.when(c == 1)
        def _():
            i_sl = iall_v.at[pl.ds(0, epw1)]
            n_sl = nall_v.at[pl.ds(0, epw1)]
            pltpu.async_copy(gidx_hbm.at[pl.ds(off0, epw1)], i_sl, semz)
            pltpu.async_copy(norm_hbm.at[pl.ds(off0, epw1)], n_sl, semz)
            pltpu.make_async_copy(gidx_hbm.at[pl.ds(off0, epw1)], i_sl,
                                  semz).wait()
            pltpu.make_async_copy(norm_hbm.at[pl.ds(off0, epw1)], n_sl,
                                  semz).wait()

        def start_gather(t, buf, sem):
            pltpu.async_copy(
                xw_hbm.at[iall_v.at[pl.ds(t * chunk, chunk)]], buf, sem)

        def wait_gather(buf, sem):
            pltpu.make_async_copy(xw_hbm.at[pl.ds(0, chunk)], buf, sem).wait()

        def start_dst(t, dbuf, sem):
            pltpu.async_copy(dst_hbm.at[pl.ds(off0 + t * chunk, chunk)],
                             dbuf, sem)

        def wait_dst(dbuf, sem):
            pltpu.make_async_copy(dst_hbm.at[pl.ds(0, chunk)], dbuf,
                                  sem).wait()

        def scale(t, buf):
            for g in range(chunk // _LANES):
                nv = nall_v[pl.ds(t * chunk + g * _LANES, _LANES)]
                for i in range(_LANES):
                    ee = g * _LANES + i
                    nb = nv[i]
                    for j in range(d // _LANES):
                        sl = pl.ds(j * _LANES, _LANES)
                        buf[ee, sl] = buf[ee, sl] * nb

        def start_scatter(buf, dbuf, sem):
            pltpu.async_copy(buf, agg_sh.at[dbuf], sem, add=True)

        def wait_scatter(buf, dbuf, sem):
            pltpu.make_async_copy(buf, agg_sh.at[dbuf], sem).wait()

        start_dst(0, dbuf0_v, semd0)
        start_dst(1, dbuf1_v, semd1)
        start_gather(0, rows0_v, semg0)
        start_gather(1, rows1_v, semg1)

        def step2(p, _):
            t0 = 2 * p
            wait_gather(rows0_v, semg0)
            scale(t0, rows0_v)
            wait_dst(dbuf0_v, semd0)
            start_scatter(rows0_v, dbuf0_v, sems0)

            wait_gather(rows1_v, semg1)
            scale(t0 + 1, rows1_v)
            wait_dst(dbuf1_v, semd1)
            start_scatter(rows1_v, dbuf1_v, sems1)

            wait_scatter(rows0_v, dbuf0_v, sems0)

            @pl.when(t0 + 2 < steps)
            def _():
                start_dst(t0 + 2, dbuf0_v, semd0)
                start_gather(t0 + 2, rows0_v, semg0)

            wait_scatter(rows1_v, dbuf1_v, sems1)

            @pl.when(t0 + 3 < steps)
            def _():
                start_dst(t0 + 3, dbuf1_v, semd1)
                start_gather(t0 + 3, rows1_v, semg1)

            return 0

        lax.fori_loop(0, steps // 2, step2, 0)
        plsc.subcore_barrier()

        dunit = 200
        dunits = n // dunit
        drounds = (dunits + _NS - 1) // _NS
        for k in range(drounds):
            u = s + k * _NS

            @pl.when(u < dunits)
            def _():
                pltpu.async_copy(agg_sh.at[pl.ds(u * dunit, dunit)],
                                 out_hbm.at[pl.ds(c * n + u * dunit, dunit)],
                                 semz)
        for k in range(drounds):
            u = s + k * _NS

            @pl.when(u < dunits)
            def _():
                pltpu.make_async_copy(agg_sh.at[pl.ds(0, dunit)],
                                      out_hbm.at[pl.ds(0, dunit)],
                                      semz).wait()

    mesh = plsc.VectorSubcoreMesh(core_axis_name="c", subcore_axis_name="s")
    return pl.kernel(
        body,
        out_type=jax.ShapeDtypeStruct((_NC * n, d), jnp.float32),
        mesh=mesh,
        scratch_types=[
            pltpu.VMEM((epw0,), jnp.int32),
            pltpu.VMEM((epw0,), jnp.float32),
            pltpu.VMEM((chunk, d), jnp.float32),
            pltpu.VMEM((chunk, d), jnp.float32),
            pltpu.VMEM((chunk,), jnp.int32),
            pltpu.VMEM((chunk,), jnp.int32),
            pltpu.VMEM_SHARED((n, d), jnp.float32),
            pltpu.SemaphoreType.DMA,
            pltpu.SemaphoreType.DMA,
            pltpu.SemaphoreType.DMA,
            pltpu.SemaphoreType.DMA,
            pltpu.SemaphoreType.DMA,
            pltpu.SemaphoreType.DMA,
            pltpu.SemaphoreType.DMA,
        ],
    )


def _bn_body(n, eps, part_ref, res_ref, bias_ref, gamma_ref, beta_ref,
             out_ref):
    agg = part_ref[:n] + part_ref[n:]
    h = jnp.maximum(agg + bias_ref[...], 0.0)
    new = h + res_ref[...]
    mean = jnp.mean(new, axis=0, keepdims=True)
    var = jnp.mean((new - mean) * (new - mean), axis=0, keepdims=True)
    inv = lax.rsqrt(var + eps)
    out_ref[...] = (new - mean) * (inv * gamma_ref[...]) + beta_ref[...]


def _combine_bn(part, res, bias, gamma, beta, n, d):
    return pl.pallas_call(
        functools.partial(_bn_body, n, _EPS),
        out_shape=jax.ShapeDtypeStruct((n, d), jnp.float32),
    )(part, res, bias.reshape(1, d), gamma.reshape(1, d), beta.reshape(1, d))


def kernel(node_feats, edge_index, etype, norm, W, bias, Wres, bres, gamma,
           beta):
    n, d_in = node_feats.shape
    nrels, _, d_out = W.shape
    e = etype.shape[0]

    src = edge_index[0]
    dst = edge_index[1]

    w_all = jnp.concatenate([W, Wres[None]], axis=0)
    xw = _compute_xw(node_feats, w_all, bres, nrels, 2000)
    xw_flat = xw.reshape((nrels + 1) * n, d_out)
    res = xw_flat[nrels * n:]

    gidx = _compute_gidx(etype, src, n)

    chunk = 64
    nw = _NC * _NS
    quantum = nw * chunk * 2
    e_pad = ((e + quantum - 1) // quantum) * quantum
    pad = e_pad - e
    gidx_p = jnp.pad(gidx, (0, pad))
    dst_p = jnp.pad(dst, (0, pad))
    norm_p = jnp.pad(norm.reshape(e), (0, pad))

    sc = _make_sc_edge_kernel(n, d_out, e_pad, chunk, 0.82)
    part = sc(gidx_p, dst_p, norm_p, xw_flat)

    return _combine_bn(part, res, bias, gamma, beta, n, d_out)
